# baseline (speedup 1.0000x reference)
import functools
import math

import jax
import jax.numpy as jnp
from jax import lax
from jax.experimental import pallas as pl
from jax.experimental.pallas import tpu as pltpu
from jax.experimental.pallas import tpu_sc as plsc

N = 10000
E = 320000
D = 128
HEADS = 8
DIM_HEAD = 64
INNER = HEADS * DIM_HEAD
LANDMARKS = 256
PINV_ITERS = 6
KERNEL = 33
WQK_DIM = 256
ATT_DIM = 128
N_CLASSES = 2

_NP = 10240
_PAD = _NP - N
_BLK = 1280
_NB = _NP // _BLK
_LPL = _NP // LANDMARKS
_LM_BLK = _BLK // _LPL
_F32 = jnp.float32


def _qkv_body(x_ref, w_ref, q3_ref, k3_ref, v3_ref, ql3_ref, kl3_ref):
    qkv = jnp.dot(x_ref[...], w_ref[...], preferred_element_type=_F32)
    q = qkv[:, :INNER] * (DIM_HEAD ** -0.5)
    k = qkv[:, INNER:2 * INNER]
    v = qkv[:, 2 * INNER:]
    qlm = q.reshape(_LM_BLK, _LPL, INNER).mean(axis=1)
    klm = k.reshape(_LM_BLK, _LPL, INNER).mean(axis=1)
    for h in range(HEADS):
        sl = slice(h * DIM_HEAD, (h + 1) * DIM_HEAD)
        q3_ref[h] = q[:, sl]
        k3_ref[h] = k[:, sl]
        v3_ref[h] = v[:, sl]
        ql3_ref[h] = qlm[:, sl]
        kl3_ref[h] = klm[:, sl]


def _qkv_landmarks(x_pad, Wqkv):
    return pl.pallas_call(
        _qkv_body,
        grid=(_NB,),
        in_specs=[
            pl.BlockSpec((_BLK, D), lambda i: (i, 0)),
            pl.BlockSpec((D, 3 * INNER), lambda i: (0, 0)),
        ],
        out_specs=[
            pl.BlockSpec((HEADS, _BLK, DIM_HEAD), lambda i: (0, i, 0)),
            pl.BlockSpec((HEADS, _BLK, DIM_HEAD), lambda i: (0, i, 0)),
            pl.BlockSpec((HEADS, _BLK, DIM_HEAD), lambda i: (0, i, 0)),
            pl.BlockSpec((HEADS, _LM_BLK, DIM_HEAD), lambda i: (0, i, 0)),
            pl.BlockSpec((HEADS, _LM_BLK, DIM_HEAD), lambda i: (0, i, 0)),
        ],
        out_shape=[
            jax.ShapeDtypeStruct((HEADS, _NP, DIM_HEAD), _F32),
            jax.ShapeDtypeStruct((HEADS, _NP, DIM_HEAD), _F32),
            jax.ShapeDtypeStruct((HEADS, _NP, DIM_HEAD), _F32),
            jax.ShapeDtypeStruct((HEADS, LANDMARKS, DIM_HEAD), _F32),
            jax.ShapeDtypeStruct((HEADS, LANDMARKS, DIM_HEAD), _F32),
        ],
    )(x_pad, Wqkv)


def _softmax_lanes(x):
    m = jnp.max(x, axis=-1, keepdims=True)
    e = jnp.exp(x - m)
    return e / jnp.sum(e, axis=-1, keepdims=True)


def _head_body(ql_ref, kl_ref, k_ref, v_ref, km_ref, m_ref, conv_ref, vpad_ref):
    h = pl.program_id(0)
    qlh = ql_ref[0]
    klh = kl_ref[0]
    kh = k_ref[0]
    vh = v_ref[0]
    contract_last = (((1,), (1,)), ((), ()))
    sim3 = lax.dot_general(qlh, kh, contract_last, preferred_element_type=_F32)
    attn3 = _softmax_lanes(sim3)
    a3v = jnp.dot(attn3, vh, preferred_element_type=_F32)
    sim2 = lax.dot_general(qlh, klh, contract_last, preferred_element_type=_F32)
    attn2 = _softmax_lanes(sim2)
    rows_i = lax.broadcasted_iota(jnp.int32, (LANDMARKS, LANDMARKS), 0)
    cols_i = lax.broadcasted_iota(jnp.int32, (LANDMARKS, LANDMARKS), 1)
    I = (rows_i == cols_i).astype(_F32)
    absx = jnp.abs(attn2)
    denom = jnp.max(jnp.sum(absx, axis=1)) * jnp.max(jnp.sum(absx, axis=0))
    xT = lax.dot_general(attn2, I, (((0,), (0,)), ((), ())),
                         preferred_element_type=_F32)
    z = xT / denom
    for _ in range(PINV_ITERS):
        xz = jnp.dot(attn2, z, preferred_element_type=_F32)
        t = 7.0 * I - xz
        t = 15.0 * I - jnp.dot(xz, t, preferred_element_type=_F32)
        t = 13.0 * I - jnp.dot(xz, t, preferred_element_type=_F32)
        z = 0.25 * jnp.dot(z, t, preferred_element_type=_F32)
    m_ref[0] = jnp.dot(z, a3v, preferred_element_type=_F32)
    vpad_ref[pl.ds(0, KERNEL // 2), :] = jnp.zeros((KERNEL // 2, DIM_HEAD), _F32)
    vpad_ref[pl.ds(KERNEL // 2, _NP), :] = vh
    vpad_ref[pl.ds(KERNEL // 2 + _NP, KERNEL // 2), :] = (
        jnp.zeros((KERNEL // 2, DIM_HEAD), _F32))
    acc = km_ref[h, 0] * vpad_ref[pl.ds(0, _NP), :]
    for t in range(1, KERNEL):
        acc = acc + km_ref[h, t] * vpad_ref[pl.ds(t, _NP), :]
    conv_ref[0] = acc


def _head_stage(ql, kl, k, v, kmat):
    return pl.pallas_call(
        _head_body,
        grid=(HEADS,),
        in_specs=[
            pl.BlockSpec((1, LANDMARKS, DIM_HEAD), lambda h: (h, 0, 0)),
            pl.BlockSpec((1, LANDMARKS, DIM_HEAD), lambda h: (h, 0, 0)),
            pl.BlockSpec((1, _NP, DIM_HEAD), lambda h: (h, 0, 0)),
            pl.BlockSpec((1, _NP, DIM_HEAD), lambda h: (h, 0, 0)),
            pl.BlockSpec(memory_space=pltpu.SMEM),
        ],
        out_specs=[
            pl.BlockSpec((1, LANDMARKS, DIM_HEAD), lambda h: (h, 0, 0)),
            pl.BlockSpec((1, _NP, DIM_HEAD), lambda h: (h, 0, 0)),
        ],
        out_shape=[
            jax.ShapeDtypeStruct((HEADS, LANDMARKS, DIM_HEAD), _F32),
            jax.ShapeDtypeStruct((HEADS, _NP, DIM_HEAD), _F32),
        ],
        scratch_shapes=[pltpu.VMEM((_NP + KERNEL - 1, DIM_HEAD), _F32)],
    )(ql, kl, k, v, kmat)


def _out_body(q_ref, kl_ref, m_ref, conv_ref, x_ref, wout_ref, bout_ref,
              wq_ref, wqb_ref, wk_ref, wkb_ref, wv_ref, wvb_ref,
              enc_ref, q2_ref, k2_ref, val_ref):
    contract_last = (((1,), (1,)), ((), ()))
    outs = []
    for h in range(HEADS):
        sim1 = lax.dot_general(q_ref[h], kl_ref[h], contract_last,
                               preferred_element_type=_F32)
        attn1 = _softmax_lanes(sim1)
        outs.append(jnp.dot(attn1, m_ref[h], preferred_element_type=_F32)
                    + conv_ref[h])
    out = jnp.concatenate(outs, axis=1)
    x = x_ref[...]
    enc = (jnp.dot(out, wout_ref[...], preferred_element_type=_F32)
           + bout_ref[...] + x)
    enc_ref[...] = enc
    q2_ref[...] = jnp.dot(enc, wq_ref[...], preferred_element_type=_F32) + wqb_ref[...]
    k2_ref[...] = jnp.dot(enc, wk_ref[...], preferred_element_type=_F32) + wkb_ref[...]
    val_ref[...] = jnp.dot(x, wv_ref[...], preferred_element_type=_F32) + wvb_ref[...]


def _out_stage(q, kl, M, conv, x_pad, Wout, bout, wq_W, wq_b, wk_W, wk_b,
               wv_W, wv_b):
    return pl.pallas_call(
        _out_body,
        grid=(_NB,),
        in_specs=[
            pl.BlockSpec((HEADS, _BLK, DIM_HEAD), lambda i: (0, i, 0)),
            pl.BlockSpec((HEADS, LANDMARKS, DIM_HEAD), lambda i: (0, 0, 0)),
            pl.BlockSpec((HEADS, LANDMARKS, DIM_HEAD), lambda i: (0, 0, 0)),
            pl.BlockSpec((HEADS, _BLK, DIM_HEAD), lambda i: (0, i, 0)),
            pl.BlockSpec((_BLK, D), lambda i: (i, 0)),
            pl.BlockSpec((INNER, D), lambda i: (0, 0)),
            pl.BlockSpec((1, D), lambda i: (0, 0)),
            pl.BlockSpec((D, WQK_DIM), lambda i: (0, 0)),
            pl.BlockSpec((1, WQK_DIM), lambda i: (0, 0)),
            pl.BlockSpec((D, WQK_DIM), lambda i: (0, 0)),
            pl.BlockSpec((1, WQK_DIM), lambda i: (0, 0)),
            pl.BlockSpec((D, D), lambda i: (0, 0)),
            pl.BlockSpec((1, D), lambda i: (0, 0)),
        ],
        out_specs=[
            pl.BlockSpec((_BLK, D), lambda i: (i, 0)),
            pl.BlockSpec((_BLK, WQK_DIM), lambda i: (i, 0)),
            pl.BlockSpec((_BLK, WQK_DIM), lambda i: (i, 0)),
            pl.BlockSpec((_BLK, D), lambda i: (i, 0)),
        ],
        out_shape=[
            jax.ShapeDtypeStruct((_NP, D), _F32),
            jax.ShapeDtypeStruct((_NP, WQK_DIM), _F32),
            jax.ShapeDtypeStruct((_NP, WQK_DIM), _F32),
            jax.ShapeDtypeStruct((_NP, D), _F32),
        ],
    )(q, kl, M, conv, x_pad, Wout, bout, wq_W, wq_b, wk_W, wk_b, wv_W, wv_b)


_SC_NC = 2
_SC_NS = 16
_SC_NW = _SC_NC * _SC_NS
_EDGE_CHUNK = 80
_EPW = E // _SC_NW
_NCHUNKS = _EPW // _EDGE_CHUNK


def _edge_sc_body(rows_hbm, cols_hbm, adj_hbm, q2_hbm, k2_hbm, out_hbm,
                  idx_r, idx_c, adj_v, qv, kv, a_loc, sem):
    wid = lax.axis_index("s") * _SC_NC + lax.axis_index("c")
    base = wid * _EPW
    lanes = lax.iota(jnp.int32, 16)

    def zero_body(i, _):
        a_loc[pl.ds(i * 16, 16)] = jnp.zeros((16,), jnp.float32)
        return 0
    lax.fori_loop(0, N // 16, zero_body, 0)

    def chunk(ci, _):
        off = base + ci * _EDGE_CHUNK
        pltpu.sync_copy(rows_hbm.at[pl.ds(off, _EDGE_CHUNK)], idx_r)
        pltpu.sync_copy(cols_hbm.at[pl.ds(off, _EDGE_CHUNK)], idx_c)
        pltpu.sync_copy(adj_hbm.at[pl.ds(off, _EDGE_CHUNK)], adj_v)
        cp_q = pltpu.async_copy(q2_hbm.at[idx_r], qv, sem)
        cp_k = pltpu.async_copy(k2_hbm.at[idx_c], kv, sem)
        cp_q.wait()
        cp_k.wait()

        for g in range(_EDGE_CHUNK // 16):
            def edge_body(e2, svec):
                e = 16 * g + e2
                acc = qv[e, pl.ds(0, 16)] * kv[e, pl.ds(0, 16)]
                for j in range(1, WQK_DIM // 16):
                    acc = acc + (qv[e, pl.ds(16 * j, 16)]
                                 * kv[e, pl.ds(16 * j, 16)])
                s = jnp.sum(acc)
                return jnp.where(lanes == e2, s, svec)
            svec = lax.fori_loop(0, 16, edge_body,
                                 jnp.zeros((16,), jnp.float32), unroll=4)
            s16 = svec * adj_v[pl.ds(16 * g, 16)] * (1.0 / 16.0)
            row_vec = idx_r[pl.ds(16 * g, 16)]
            for j in range(16):
                plsc.addupdate_scatter(a_loc, [row_vec], s16,
                                       mask=lanes == j)
        return 0
    lax.fori_loop(0, _NCHUNKS, chunk, 0)
    pltpu.sync_copy(a_loc, out_hbm.at[wid])


def _edge_scores_sc(rows, cols, adj, q2, k2):
    mesh = plsc.VectorSubcoreMesh(core_axis_name="c", subcore_axis_name="s",
                                  num_cores=_SC_NC, num_subcores=_SC_NS)
    f = pl.kernel(
        _edge_sc_body,
        out_type=jax.ShapeDtypeStruct((_SC_NW, N), jnp.float32),
        mesh=mesh,
        compiler_params=pltpu.CompilerParams(use_tc_tiling_on_sc=False,
                                             needs_layout_passes=False),
        scratch_types=[
            pltpu.VMEM((_EDGE_CHUNK,), jnp.int32),
            pltpu.VMEM((_EDGE_CHUNK,), jnp.int32),
            pltpu.VMEM((_EDGE_CHUNK,), jnp.float32),
            pltpu.VMEM((_EDGE_CHUNK, WQK_DIM), jnp.float32),
            pltpu.VMEM((_EDGE_CHUNK, WQK_DIM), jnp.float32),
            pltpu.VMEM((N,), jnp.float32),
            pltpu.SemaphoreType.DMA,
        ],
    )
    return f(rows, cols, adj, q2, k2)


def _epilogue_body(p_ref, val_ref, enc_ref, vw_ref, vb_ref, uw_ref, ub_ref,
                   ww_ref, wb_ref, fc_ref, fcb_ref, out_ref):
    A = jnp.sum(p_ref[...], axis=1, keepdims=True)
    A = A - jnp.max(A)
    ea = jnp.exp(A)
    alpha = ea / jnp.sum(ea)
    xl = alpha * val_ref[...]
    wei = jax.nn.sigmoid(-xl)
    sw = wei * wei
    xo = xl * 2.0 * sw + 2.0 * enc_ref[...] * (1.0 - sw)
    inst = jnp.tanh(jnp.dot(xo, vw_ref[...], preferred_element_type=_F32)
                    + vb_ref[...])
    gate = jax.nn.sigmoid(jnp.dot(xo, uw_ref[...], preferred_element_type=_F32)
                          + ub_ref[...])
    scores = (jnp.dot(inst * gate, ww_ref[...], preferred_element_type=_F32)
              + wb_ref[...])
    scores = scores - jnp.max(scores)
    es = jnp.exp(scores)
    ka = es / jnp.sum(es)
    pooled = jnp.sum(ka * xo, axis=0, keepdims=True) * (1.0 / N)
    out_ref[...] = (jnp.dot(pooled, fc_ref[...], preferred_element_type=_F32)
                    + fcb_ref[...])


def _epilogue(partials_t, value, enc, v_W, v_b, u_W, u_b, w_W, w_b, fc_all):
    return pl.pallas_call(
        _epilogue_body,
        out_shape=jax.ShapeDtypeStruct((1, N_CLASSES), _F32),
    )(partials_t, value, enc, v_W, v_b[None, :], u_W, u_b[None, :],
      w_W, w_b[None, :], fc_all[0], fc_all[1])


def kernel(dense, edge_index, adj_values, Wqkv, Wout, bout, res_kernel,
           wq_W, wq_b, wk_W, wk_b, wv_W, wv_b,
           v_W, v_b, u_W, u_b, w_W, w_b,
           fc_W, fc_b, fc_bias):
    x_pad = jnp.pad(dense[0], ((_PAD, 0), (0, 0)))
    q3, k3, v3, ql3, kl3 = _qkv_landmarks(x_pad, Wqkv)
    kmat = res_kernel[:, 0, :, 0]
    M, conv3 = _head_stage(ql3, kl3, k3, v3, kmat)
    enc_f, q2_f, k2_f, val_f = _out_stage(
        q3, kl3, M, conv3, x_pad, Wout, bout[None, :],
        wq_W, wq_b[None, :], wk_W, wk_b[None, :], wv_W, wv_b[None, :])
    enc = enc_f[_PAD:]
    q2 = q2_f[_PAD:]
    k2 = k2_f[_PAD:]
    value = val_f[_PAD:]
    partials = _edge_scores_sc(edge_index[0], edge_index[1], adj_values,
                               q2, k2)
    fc_all = (fc_W, (fc_b + fc_bias)[None, :])
    return _epilogue(partials.T, value, enc, v_W, v_b, u_W, u_b, w_W, w_b,
                     fc_all)


# speedup vs baseline: 1.4358x; 1.4358x over previous
import functools
import math

import jax
import jax.numpy as jnp
from jax import lax
from jax.experimental import pallas as pl
from jax.experimental.pallas import tpu as pltpu
from jax.experimental.pallas import tpu_sc as plsc

N = 10000
E = 320000
D = 128
HEADS = 8
DIM_HEAD = 64
INNER = HEADS * DIM_HEAD
LANDMARKS = 256
PINV_ITERS = 6
KERNEL = 33
WQK_DIM = 256
ATT_DIM = 128
N_CLASSES = 2

_NP = 10240
_PAD = _NP - N
_BLK = 1280
_NB = _NP // _BLK
_LPL = _NP // LANDMARKS
_LM_BLK = _BLK // _LPL
_F32 = jnp.float32


def _qkv_body(x_ref, w_ref, q3_ref, k3_ref, v3_ref, ql3_ref, kl3_ref):
    qkv = jnp.dot(x_ref[...], w_ref[...], preferred_element_type=_F32)
    q = qkv[:, :INNER] * (DIM_HEAD ** -0.5)
    k = qkv[:, INNER:2 * INNER]
    v = qkv[:, 2 * INNER:]
    qlm = q.reshape(_LM_BLK, _LPL, INNER).mean(axis=1)
    klm = k.reshape(_LM_BLK, _LPL, INNER).mean(axis=1)
    for h in range(HEADS):
        sl = slice(h * DIM_HEAD, (h + 1) * DIM_HEAD)
        q3_ref[h] = q[:, sl]
        k3_ref[h] = k[:, sl]
        v3_ref[h] = v[:, sl]
        ql3_ref[h] = qlm[:, sl]
        kl3_ref[h] = klm[:, sl]


def _qkv_landmarks(x_pad, Wqkv):
    return pl.pallas_call(
        _qkv_body,
        grid=(_NB,),
        in_specs=[
            pl.BlockSpec((_BLK, D), lambda i: (i, 0)),
            pl.BlockSpec((D, 3 * INNER), lambda i: (0, 0)),
        ],
        out_specs=[
            pl.BlockSpec((HEADS, _BLK, DIM_HEAD), lambda i: (0, i, 0)),
            pl.BlockSpec((HEADS, _BLK, DIM_HEAD), lambda i: (0, i, 0)),
            pl.BlockSpec((HEADS, _BLK, DIM_HEAD), lambda i: (0, i, 0)),
            pl.BlockSpec((HEADS, _LM_BLK, DIM_HEAD), lambda i: (0, i, 0)),
            pl.BlockSpec((HEADS, _LM_BLK, DIM_HEAD), lambda i: (0, i, 0)),
        ],
        out_shape=[
            jax.ShapeDtypeStruct((HEADS, _NP, DIM_HEAD), _F32),
            jax.ShapeDtypeStruct((HEADS, _NP, DIM_HEAD), _F32),
            jax.ShapeDtypeStruct((HEADS, _NP, DIM_HEAD), _F32),
            jax.ShapeDtypeStruct((HEADS, LANDMARKS, DIM_HEAD), _F32),
            jax.ShapeDtypeStruct((HEADS, LANDMARKS, DIM_HEAD), _F32),
        ],
    )(x_pad, Wqkv)


def _softmax_lanes(x):
    m = jnp.max(x, axis=-1, keepdims=True)
    e = jnp.exp(x - m)
    return e / jnp.sum(e, axis=-1, keepdims=True)


def _head_body(ql_ref, kl_ref, k_ref, v_ref, km_ref, m_ref, conv_ref, vpad_ref):
    h = pl.program_id(0)
    qlh = ql_ref[0]
    klh = kl_ref[0]
    kh = k_ref[0]
    vh = v_ref[0]
    contract_last = (((1,), (1,)), ((), ()))
    sim3 = lax.dot_general(qlh, kh, contract_last, preferred_element_type=_F32)
    attn3 = _softmax_lanes(sim3)
    a3v = jnp.dot(attn3, vh, preferred_element_type=_F32)
    sim2 = lax.dot_general(qlh, klh, contract_last, preferred_element_type=_F32)
    attn2 = _softmax_lanes(sim2)
    rows_i = lax.broadcasted_iota(jnp.int32, (LANDMARKS, LANDMARKS), 0)
    cols_i = lax.broadcasted_iota(jnp.int32, (LANDMARKS, LANDMARKS), 1)
    I = (rows_i == cols_i).astype(_F32)
    absx = jnp.abs(attn2)
    denom = jnp.max(jnp.sum(absx, axis=1)) * jnp.max(jnp.sum(absx, axis=0))
    xT = lax.dot_general(attn2, I, (((0,), (0,)), ((), ())),
                         preferred_element_type=_F32)
    z = xT / denom
    for _ in range(PINV_ITERS):
        xz = jnp.dot(attn2, z, preferred_element_type=_F32)
        t = 7.0 * I - xz
        t = 15.0 * I - jnp.dot(xz, t, preferred_element_type=_F32)
        t = 13.0 * I - jnp.dot(xz, t, preferred_element_type=_F32)
        z = 0.25 * jnp.dot(z, t, preferred_element_type=_F32)
    m_ref[0] = jnp.dot(z, a3v, preferred_element_type=_F32)
    vpad_ref[pl.ds(0, KERNEL // 2), :] = jnp.zeros((KERNEL // 2, DIM_HEAD), _F32)
    vpad_ref[pl.ds(KERNEL // 2, _NP), :] = vh
    vpad_ref[pl.ds(KERNEL // 2 + _NP, KERNEL // 2), :] = (
        jnp.zeros((KERNEL // 2, DIM_HEAD), _F32))
    acc = km_ref[h, 0] * vpad_ref[pl.ds(0, _NP), :]
    for t in range(1, KERNEL):
        acc = acc + km_ref[h, t] * vpad_ref[pl.ds(t, _NP), :]
    conv_ref[0] = acc


def _head_stage(ql, kl, k, v, kmat):
    return pl.pallas_call(
        _head_body,
        grid=(HEADS,),
        in_specs=[
            pl.BlockSpec((1, LANDMARKS, DIM_HEAD), lambda h: (h, 0, 0)),
            pl.BlockSpec((1, LANDMARKS, DIM_HEAD), lambda h: (h, 0, 0)),
            pl.BlockSpec((1, _NP, DIM_HEAD), lambda h: (h, 0, 0)),
            pl.BlockSpec((1, _NP, DIM_HEAD), lambda h: (h, 0, 0)),
            pl.BlockSpec(memory_space=pltpu.SMEM),
        ],
        out_specs=[
            pl.BlockSpec((1, LANDMARKS, DIM_HEAD), lambda h: (h, 0, 0)),
            pl.BlockSpec((1, _NP, DIM_HEAD), lambda h: (h, 0, 0)),
        ],
        out_shape=[
            jax.ShapeDtypeStruct((HEADS, LANDMARKS, DIM_HEAD), _F32),
            jax.ShapeDtypeStruct((HEADS, _NP, DIM_HEAD), _F32),
        ],
        scratch_shapes=[pltpu.VMEM((_NP + KERNEL - 1, DIM_HEAD), _F32)],
    )(ql, kl, k, v, kmat)


def _out_body(q_ref, kl_ref, m_ref, conv_ref, x_ref, wout_ref, bout_ref,
              wq_ref, wqb_ref, wk_ref, wkb_ref, wv_ref, wvb_ref,
              enc_ref, q2_ref, k2_ref, val_ref):
    contract_last = (((1,), (1,)), ((), ()))
    outs = []
    for h in range(HEADS):
        sim1 = lax.dot_general(q_ref[h], kl_ref[h], contract_last,
                               preferred_element_type=_F32)
        attn1 = _softmax_lanes(sim1)
        outs.append(jnp.dot(attn1, m_ref[h], preferred_element_type=_F32)
                    + conv_ref[h])
    out = jnp.concatenate(outs, axis=1)
    x = x_ref[...]
    enc = (jnp.dot(out, wout_ref[...], preferred_element_type=_F32)
           + bout_ref[...] + x)
    enc_ref[...] = enc
    q2_ref[...] = jnp.dot(enc, wq_ref[...], preferred_element_type=_F32) + wqb_ref[...]
    k2_ref[...] = jnp.dot(enc, wk_ref[...], preferred_element_type=_F32) + wkb_ref[...]
    val_ref[...] = jnp.dot(x, wv_ref[...], preferred_element_type=_F32) + wvb_ref[...]


def _out_stage(q, kl, M, conv, x_pad, Wout, bout, wq_W, wq_b, wk_W, wk_b,
               wv_W, wv_b):
    return pl.pallas_call(
        _out_body,
        grid=(_NB,),
        in_specs=[
            pl.BlockSpec((HEADS, _BLK, DIM_HEAD), lambda i: (0, i, 0)),
            pl.BlockSpec((HEADS, LANDMARKS, DIM_HEAD), lambda i: (0, 0, 0)),
            pl.BlockSpec((HEADS, LANDMARKS, DIM_HEAD), lambda i: (0, 0, 0)),
            pl.BlockSpec((HEADS, _BLK, DIM_HEAD), lambda i: (0, i, 0)),
            pl.BlockSpec((_BLK, D), lambda i: (i, 0)),
            pl.BlockSpec((INNER, D), lambda i: (0, 0)),
            pl.BlockSpec((1, D), lambda i: (0, 0)),
            pl.BlockSpec((D, WQK_DIM), lambda i: (0, 0)),
            pl.BlockSpec((1, WQK_DIM), lambda i: (0, 0)),
            pl.BlockSpec((D, WQK_DIM), lambda i: (0, 0)),
            pl.BlockSpec((1, WQK_DIM), lambda i: (0, 0)),
            pl.BlockSpec((D, D), lambda i: (0, 0)),
            pl.BlockSpec((1, D), lambda i: (0, 0)),
        ],
        out_specs=[
            pl.BlockSpec((_BLK, D), lambda i: (i, 0)),
            pl.BlockSpec((_BLK, WQK_DIM), lambda i: (i, 0)),
            pl.BlockSpec((_BLK, WQK_DIM), lambda i: (i, 0)),
            pl.BlockSpec((_BLK, D), lambda i: (i, 0)),
        ],
        out_shape=[
            jax.ShapeDtypeStruct((_NP, D), _F32),
            jax.ShapeDtypeStruct((_NP, WQK_DIM), _F32),
            jax.ShapeDtypeStruct((_NP, WQK_DIM), _F32),
            jax.ShapeDtypeStruct((_NP, D), _F32),
        ],
    )(q, kl, M, conv, x_pad, Wout, bout, wq_W, wq_b, wk_W, wk_b, wv_W, wv_b)


_SC_NC = 2
_SC_NS = 16
_SC_NW = _SC_NC * _SC_NS
_EDGE_CHUNK = 80
_EPW = E // _SC_NW
_NCHUNKS = _EPW // _EDGE_CHUNK


def _edge_sc_body(rows_hbm, cols_hbm, adj_hbm, q2_hbm, k2_hbm, out_hbm,
                  idx_r0, idx_c0, adj_v0, qv0, kv0,
                  idx_r1, idx_c1, adj_v1, qv1, kv1,
                  a_loc, sem0, sem1):
    wid = lax.axis_index("s") * _SC_NC + lax.axis_index("c")
    base = wid * _EPW
    lanes = lax.iota(jnp.int32, 16)
    bufs = ((idx_r0, idx_c0, adj_v0, qv0, kv0, sem0),
            (idx_r1, idx_c1, adj_v1, qv1, kv1, sem1))

    def zero_body(i, _):
        a_loc[pl.ds(i * 16, 16)] = jnp.zeros((16,), jnp.float32)
        return 0
    lax.fori_loop(0, N // 16, zero_body, 0)

    def issue(ci, b):
        idx_r, idx_c, adj_v, qv, kv, sem = bufs[b]
        off = base + ci * _EDGE_CHUNK
        pltpu.sync_copy(rows_hbm.at[pl.ds(off, _EDGE_CHUNK)], idx_r)
        pltpu.sync_copy(cols_hbm.at[pl.ds(off, _EDGE_CHUNK)], idx_c)
        pltpu.sync_copy(adj_hbm.at[pl.ds(off, _EDGE_CHUNK)], adj_v)
        pltpu.async_copy(q2_hbm.at[idx_r], qv, sem)
        pltpu.async_copy(k2_hbm.at[idx_c], kv, sem)

    def consume(b):
        idx_r, idx_c, adj_v, qv, kv, sem = bufs[b]
        pltpu.make_async_copy(q2_hbm.at[idx_r], qv, sem).wait()
        pltpu.make_async_copy(k2_hbm.at[idx_c], kv, sem).wait()
        for g in range(_EDGE_CHUNK // 16):
            def edge_body(e2, svec):
                e = 16 * g + e2
                acc = qv[e, pl.ds(0, 16)] * kv[e, pl.ds(0, 16)]
                for j in range(1, WQK_DIM // 16):
                    acc = acc + (qv[e, pl.ds(16 * j, 16)]
                                 * kv[e, pl.ds(16 * j, 16)])
                s = jnp.sum(acc)
                return jnp.where(lanes == e2, s, svec)
            svec = lax.fori_loop(0, 16, edge_body,
                                 jnp.zeros((16,), jnp.float32))
            s16 = svec * adj_v[pl.ds(16 * g, 16)] * (1.0 / 16.0)
            row_vec = idx_r[pl.ds(16 * g, 16)]
            for j in range(16):
                plsc.addupdate_scatter(a_loc, [row_vec], s16,
                                       mask=lanes == j)

    issue(0, 0)

    def pair(cc, _):
        for b in range(2):
            ci = cc * 2 + b

            @pl.when(ci < _NCHUNKS)
            def _():
                @pl.when(ci + 1 < _NCHUNKS)
                def _():
                    issue(ci + 1, 1 - b)
                consume(b)
        return 0
    lax.fori_loop(0, (_NCHUNKS + 1) // 2, pair, 0)
    pltpu.sync_copy(a_loc, out_hbm.at[wid])


def _edge_scores_sc(rows, cols, adj, q2, k2):
    mesh = plsc.VectorSubcoreMesh(core_axis_name="c", subcore_axis_name="s",
                                  num_cores=_SC_NC, num_subcores=_SC_NS)
    f = pl.kernel(
        _edge_sc_body,
        out_type=jax.ShapeDtypeStruct((_SC_NW, N), jnp.float32),
        mesh=mesh,
        compiler_params=pltpu.CompilerParams(use_tc_tiling_on_sc=False,
                                             needs_layout_passes=False),
        scratch_types=[
            pltpu.VMEM((_EDGE_CHUNK,), jnp.int32),
            pltpu.VMEM((_EDGE_CHUNK,), jnp.int32),
            pltpu.VMEM((_EDGE_CHUNK,), jnp.float32),
            pltpu.VMEM((_EDGE_CHUNK, WQK_DIM), jnp.float32),
            pltpu.VMEM((_EDGE_CHUNK, WQK_DIM), jnp.float32),
            pltpu.VMEM((_EDGE_CHUNK,), jnp.int32),
            pltpu.VMEM((_EDGE_CHUNK,), jnp.int32),
            pltpu.VMEM((_EDGE_CHUNK,), jnp.float32),
            pltpu.VMEM((_EDGE_CHUNK, WQK_DIM), jnp.float32),
            pltpu.VMEM((_EDGE_CHUNK, WQK_DIM), jnp.float32),
            pltpu.VMEM((N,), jnp.float32),
            pltpu.SemaphoreType.DMA,
            pltpu.SemaphoreType.DMA,
        ],
    )
    return f(rows, cols, adj, q2, k2)


def _epilogue_body(p_ref, val_ref, enc_ref, vw_ref, vb_ref, uw_ref, ub_ref,
                   ww_ref, wb_ref, fc_ref, fcb_ref, out_ref):
    A = jnp.sum(p_ref[...], axis=1, keepdims=True)
    A = A - jnp.max(A)
    ea = jnp.exp(A)
    alpha = ea / jnp.sum(ea)
    xl = alpha * val_ref[...]
    wei = jax.nn.sigmoid(-xl)
    sw = wei * wei
    xo = xl * 2.0 * sw + 2.0 * enc_ref[...] * (1.0 - sw)
    inst = jnp.tanh(jnp.dot(xo, vw_ref[...], preferred_element_type=_F32)
                    + vb_ref[...])
    gate = jax.nn.sigmoid(jnp.dot(xo, uw_ref[...], preferred_element_type=_F32)
                          + ub_ref[...])
    scores = (jnp.dot(inst * gate, ww_ref[...], preferred_element_type=_F32)
              + wb_ref[...])
    scores = scores - jnp.max(scores)
    es = jnp.exp(scores)
    ka = es / jnp.sum(es)
    pooled = jnp.sum(ka * xo, axis=0, keepdims=True) * (1.0 / N)
    out_ref[...] = (jnp.dot(pooled, fc_ref[...], preferred_element_type=_F32)
                    + fcb_ref[...])


def _epilogue(partials_t, value, enc, v_W, v_b, u_W, u_b, w_W, w_b, fc_all):
    return pl.pallas_call(
        _epilogue_body,
        out_shape=jax.ShapeDtypeStruct((1, N_CLASSES), _F32),
    )(partials_t, value, enc, v_W, v_b[None, :], u_W, u_b[None, :],
      w_W, w_b[None, :], fc_all[0], fc_all[1])


def kernel(dense, edge_index, adj_values, Wqkv, Wout, bout, res_kernel,
           wq_W, wq_b, wk_W, wk_b, wv_W, wv_b,
           v_W, v_b, u_W, u_b, w_W, w_b,
           fc_W, fc_b, fc_bias):
    x_pad = jnp.pad(dense[0], ((_PAD, 0), (0, 0)))
    q3, k3, v3, ql3, kl3 = _qkv_landmarks(x_pad, Wqkv)
    kmat = res_kernel[:, 0, :, 0]
    M, conv3 = _head_stage(ql3, kl3, k3, v3, kmat)
    enc_f, q2_f, k2_f, val_f = _out_stage(
        q3, kl3, M, conv3, x_pad, Wout, bout[None, :],
        wq_W, wq_b[None, :], wk_W, wk_b[None, :], wv_W, wv_b[None, :])
    enc = enc_f[_PAD:]
    q2 = q2_f[_PAD:]
    k2 = k2_f[_PAD:]
    value = val_f[_PAD:]
    partials = _edge_scores_sc(edge_index[0], edge_index[1], adj_values,
                               q2, k2)
    fc_all = (fc_W, (fc_b + fc_bias)[None, :])
    return _epilogue(partials.T, value, enc, v_W, v_b, u_W, u_b, w_W, w_b,
                     fc_all)
